# Initial kernel scaffold; baseline (speedup 1.0000x reference)
import functools

import jax
import jax.numpy as jnp
from jax import lax
from jax.experimental import pallas as pl
from jax.experimental.pallas import tpu as pltpu
from jax.experimental.pallas import tpu_sc as plsc

N_NODES = 10000
E_EDGES = 160000
CH = 128
NCHUNKS = E_EDGES // CH
NUM_SC = 2
NUM_SUB = 16
NTILES = NUM_SC * NUM_SUB
ROWS_PER_SUB = N_NODES // NUM_SUB


def _matmul_body(x_ref, w_ref, o_ref):
    o_ref[...] = jnp.dot(x_ref[...], w_ref[...],
                         preferred_element_type=jnp.float32)


def _tc_matmul(x, w, bm):
    m, k = x.shape
    _, n = w.shape
    return pl.pallas_call(
        _matmul_body,
        grid=(m // bm,),
        in_specs=[pl.BlockSpec((bm, k), lambda i: (i, 0)),
                  pl.BlockSpec((k, n), lambda i: (0, 0))],
        out_specs=pl.BlockSpec((bm, n), lambda i: (i, 0)),
        out_shape=jax.ShapeDtypeStruct((m, n), jnp.float32),
    )(x, w)


def _fused_body(p0_ref, p1_ref, w_ref, o_ref):
    r = jnp.maximum(p0_ref[...] + p1_ref[...], 0.0)
    o_ref[...] = jnp.dot(r, w_ref[...], preferred_element_type=jnp.float32)


def _tc_add_relu_matmul(p0, p1, w, bm):
    m, k = p0.shape
    _, n = w.shape
    return pl.pallas_call(
        _fused_body,
        grid=(m // bm,),
        in_specs=[pl.BlockSpec((bm, k), lambda i: (i, 0)),
                  pl.BlockSpec((bm, k), lambda i: (i, 0)),
                  pl.BlockSpec((k, n), lambda i: (0, 0))],
        out_specs=pl.BlockSpec((bm, n), lambda i: (i, 0)),
        out_shape=jax.ShapeDtypeStruct((m, n), jnp.float32),
    )(p0, p1, w)


def _add_body(a_ref, b_ref, o_ref):
    o_ref[...] = a_ref[...] + b_ref[...]


def _tc_add(a, b):
    m, n = a.shape
    return pl.pallas_call(
        _add_body,
        out_shape=jax.ShapeDtypeStruct((m, n), jnp.float32),
    )(a, b)


def _make_spmm(d):
    n_iters = (NCHUNKS + NTILES - 1) // NTILES
    nvec = d // 16
    nfull = ROWS_PER_SUB // CH
    rem = ROWS_PER_SUB - nfull * CH
    mesh = plsc.VectorSubcoreMesh(core_axis_name="c", subcore_axis_name="s")

    @functools.partial(
        pl.kernel,
        out_type=jax.ShapeDtypeStruct((NUM_SC, N_NODES, d), jnp.float32),
        mesh=mesh,
        scratch_types=[
            pltpu.VMEM((1, CH), jnp.int32),
            pltpu.VMEM((1, CH), jnp.int32),
            pltpu.VMEM((1, CH), jnp.float32),
            pltpu.VMEM((CH, d), jnp.float32),
            pltpu.VMEM_SHARED((N_NODES, d), jnp.float32),
        ],
    )
    def spmm(h_hbm, dst_hbm, src_hbm, vals_hbm, out_hbm,
             srcv, dstv, valsv, rowsv, acc):
        cid = lax.axis_index("c")
        sid = lax.axis_index("s")
        wid = sid * NUM_SC + cid

        zero = jnp.zeros((16,), jnp.float32)

        @pl.loop(0, CH)
        def _(i):
            for j in range(nvec):
                rowsv[i, pl.ds(j * 16, 16)] = zero

        base = sid * ROWS_PER_SUB
        for k in range(nfull):
            pltpu.sync_copy(rowsv, acc.at[pl.ds(base + k * CH, CH)])
        if rem:
            pltpu.sync_copy(rowsv.at[pl.ds(0, rem)],
                            acc.at[pl.ds(base + nfull * CH, rem)])
        plsc.subcore_barrier()

        @pl.loop(0, n_iters)
        def _(it):
            c = wid + it * NTILES

            @pl.when(c < NCHUNKS)
            def _():
                e0 = c * CH
                pltpu.sync_copy(dst_hbm.at[pl.ds(e0, CH)], dstv.at[0])
                pltpu.sync_copy(src_hbm.at[pl.ds(e0, CH)], srcv.at[0])
                pltpu.sync_copy(vals_hbm.at[pl.ds(e0, CH)], valsv.at[0])
                pltpu.sync_copy(h_hbm.at[srcv.at[0]], rowsv)

                @pl.loop(0, CH)
                def _(e):
                    bc = jnp.full((16,), valsv[0, e], jnp.float32)
                    for j in range(nvec):
                        sl = pl.ds(j * 16, 16)
                        rowsv[e, sl] = rowsv[e, sl] * bc

                pltpu.sync_copy(rowsv, acc.at[dstv.at[0]], add=True)

        plsc.subcore_barrier()
        pltpu.sync_copy(acc.at[pl.ds(base, ROWS_PER_SUB)],
                        out_hbm.at[cid, pl.ds(base, ROWS_PER_SUB)])

    return spmm


_spmm128 = _make_spmm(128)
_spmm64 = _make_spmm(64)


def kernel(x, edge_index, adj_values, W1, W2):
    dst = edge_index[0].astype(jnp.int32)
    src = edge_index[1].astype(jnp.int32)
    h1 = _tc_matmul(x, W1, bm=2500)
    p = _spmm128(h1, dst, src, adj_values)
    h2 = _tc_add_relu_matmul(p[0], p[1], W2, bm=2500)
    q = _spmm64(h2, dst, src, adj_values)
    return _tc_add(q[0], q[1])


# speedup vs baseline: 4.2845x; 4.2845x over previous
import dataclasses
import functools

import jax
import jax.numpy as jnp
from jax import lax
from jax.experimental import pallas as pl
from jax.experimental.pallas import tpu as pltpu
from jax.experimental.pallas import tpu_sc as plsc

N_NODES = 10000
E_EDGES = 160000
CH = 128
NCHUNKS = E_EDGES // CH
NUM_SC = 2
NUM_SUB = 16
NTILES = NUM_SC * NUM_SUB
ROWS_PER_SUB = N_NODES // NUM_SUB


def _matmul_body(x_ref, w_ref, o_ref):
    o_ref[...] = jnp.dot(x_ref[...], w_ref[...],
                         preferred_element_type=jnp.float32)


def _tc_matmul(x, w, bm):
    m, k = x.shape
    _, n = w.shape
    return pl.pallas_call(
        _matmul_body,
        grid=(m // bm,),
        in_specs=[pl.BlockSpec((bm, k), lambda i: (i, 0)),
                  pl.BlockSpec((k, n), lambda i: (0, 0))],
        out_specs=pl.BlockSpec((bm, n), lambda i: (i, 0)),
        out_shape=jax.ShapeDtypeStruct((m, n), jnp.float32),
    )(x, w)


def _fused_body(p0_ref, p1_ref, w_ref, o_ref):
    r = jnp.maximum(p0_ref[...] + p1_ref[...], 0.0)
    o_ref[...] = jnp.dot(r, w_ref[...], preferred_element_type=jnp.float32)


def _tc_add_relu_matmul(p0, p1, w, bm):
    m, k = p0.shape
    _, n = w.shape
    return pl.pallas_call(
        _fused_body,
        grid=(m // bm,),
        in_specs=[pl.BlockSpec((bm, k), lambda i: (i, 0)),
                  pl.BlockSpec((bm, k), lambda i: (i, 0)),
                  pl.BlockSpec((k, n), lambda i: (0, 0))],
        out_specs=pl.BlockSpec((bm, n), lambda i: (i, 0)),
        out_shape=jax.ShapeDtypeStruct((m, n), jnp.float32),
    )(p0, p1, w)


def _add_body(a_ref, b_ref, o_ref):
    o_ref[...] = a_ref[...] + b_ref[...]


def _tc_add(a, b):
    m, n = a.shape
    return pl.pallas_call(
        _add_body,
        out_shape=jax.ShapeDtypeStruct((m, n), jnp.float32),
    )(a, b)


def _make_spmm(d):
    n_iters = (NCHUNKS + NTILES - 1) // NTILES
    nvec = d // 16
    rps = 624
    tail = N_NODES - rps * NUM_SUB
    nfull = rps // CH
    rem = rps - nfull * CH
    mesh = plsc.VectorSubcoreMesh(core_axis_name="c", subcore_axis_name="s")
    cp = pltpu.CompilerParams(needs_layout_passes=False,
                              use_tc_tiling_on_sc=False)

    @functools.partial(
        pl.kernel,
        compiler_params=cp,
        out_type=jax.ShapeDtypeStruct((NUM_SC, N_NODES, d), jnp.float32),
        mesh=mesh,
        scratch_types=[
            pltpu.VMEM((1, CH), jnp.int32),
            pltpu.VMEM((1, CH), jnp.int32),
            pltpu.VMEM((1, CH), jnp.float32),
            pltpu.VMEM((CH, d), jnp.float32),
            pltpu.VMEM_SHARED((N_NODES, d), jnp.float32),
        ],
    )
    def spmm(h_hbm, dst_hbm, src_hbm, vals_hbm, out_hbm,
             srcv, dstv, valsv, rowsv, acc):
        cid = lax.axis_index("c")
        sid = lax.axis_index("s")
        wid = sid * NUM_SC + cid

        zero = jnp.zeros((16,), jnp.float32)

        @pl.loop(0, CH)
        def _(i):
            for j in range(nvec):
                rowsv[i, pl.ds(j * 16, 16)] = zero

        base = sid * rps
        for k in range(nfull):
            pltpu.sync_copy(rowsv, acc.at[pl.ds(base + k * CH, CH)])
        if rem:
            pltpu.sync_copy(rowsv.at[pl.ds(0, rem)],
                            acc.at[pl.ds(base + nfull * CH, rem)])

        @pl.when(sid == NUM_SUB - 1)
        def _():
            pltpu.sync_copy(rowsv.at[pl.ds(0, tail)],
                            acc.at[pl.ds(rps * NUM_SUB, tail)])

        plsc.subcore_barrier()

        @pl.loop(0, n_iters)
        def _(it):
            c = wid + it * NTILES

            @pl.when(c < NCHUNKS)
            def _():
                e0 = c * CH
                pltpu.sync_copy(dst_hbm.at[pl.ds(e0, CH)], dstv.at[0])
                pltpu.sync_copy(src_hbm.at[pl.ds(e0, CH)], srcv.at[0])
                pltpu.sync_copy(vals_hbm.at[pl.ds(e0, CH)], valsv.at[0])
                pltpu.sync_copy(h_hbm.at[srcv.at[0]], rowsv)

                zidx = jnp.zeros((16,), jnp.int32)

                @pl.loop(0, CH)
                def _(e):
                    bc = plsc.load_gather(
                        valsv, [zidx, jnp.full((16,), e, jnp.int32)])
                    for j in range(nvec):
                        sl = pl.ds(j * 16, 16)
                        rowsv[e, sl] = rowsv[e, sl] * bc

                pltpu.sync_copy(rowsv, acc.at[dstv.at[0]], add=True)

        plsc.subcore_barrier()
        pltpu.sync_copy(acc.at[pl.ds(base, rps)],
                        out_hbm.at[cid, pl.ds(base, rps)])

        @pl.when(sid == NUM_SUB - 1)
        def _():
            pltpu.sync_copy(acc.at[pl.ds(rps * NUM_SUB, tail)],
                            out_hbm.at[cid, pl.ds(rps * NUM_SUB, tail)])

    return spmm


_spmm128 = _make_spmm(128)
_spmm64 = _make_spmm(64)


def kernel(x, edge_index, adj_values, W1, W2):
    dst = edge_index[0].astype(jnp.int32)
    src = edge_index[1].astype(jnp.int32)
    h1 = _tc_matmul(x, W1, bm=2000)
    p = _spmm128(h1, dst, src, adj_values)
    h2 = _tc_add_relu_matmul(p[0], p[1], W2, bm=2000)
    q = _spmm64(h2, dst, src, adj_values)
    return _tc_add(q[0], q[1])


# speedup vs baseline: 6.6024x; 1.5410x over previous
import dataclasses
import functools

import jax
import jax.numpy as jnp
from jax import lax
from jax.experimental import pallas as pl
from jax.experimental.pallas import tpu as pltpu
from jax.experimental.pallas import tpu_sc as plsc

N_NODES = 10000
E_EDGES = 160000
CH = 128
NCHUNKS = E_EDGES // CH
NUM_SC = 2
NUM_SUB = 16
NTILES = NUM_SC * NUM_SUB
ROWS_PER_SUB = N_NODES // NUM_SUB


def _matmul_body(x_ref, w_ref, o_ref):
    o_ref[...] = jnp.dot(x_ref[...], w_ref[...],
                         preferred_element_type=jnp.float32)


def _tc_matmul(x, w, bm):
    m, k = x.shape
    _, n = w.shape
    return pl.pallas_call(
        _matmul_body,
        grid=(m // bm,),
        in_specs=[pl.BlockSpec((bm, k), lambda i: (i, 0)),
                  pl.BlockSpec((k, n), lambda i: (0, 0))],
        out_specs=pl.BlockSpec((bm, n), lambda i: (i, 0)),
        out_shape=jax.ShapeDtypeStruct((m, n), jnp.float32),
    )(x, w)


def _fused_body(p0_ref, p1_ref, w_ref, o_ref):
    r = jnp.maximum(p0_ref[...] + p1_ref[...], 0.0)
    o_ref[...] = jnp.dot(r, w_ref[...], preferred_element_type=jnp.float32)


def _tc_add_relu_matmul(p0, p1, w, bm):
    m, k = p0.shape
    _, n = w.shape
    return pl.pallas_call(
        _fused_body,
        grid=(m // bm,),
        in_specs=[pl.BlockSpec((bm, k), lambda i: (i, 0)),
                  pl.BlockSpec((bm, k), lambda i: (i, 0)),
                  pl.BlockSpec((k, n), lambda i: (0, 0))],
        out_specs=pl.BlockSpec((bm, n), lambda i: (i, 0)),
        out_shape=jax.ShapeDtypeStruct((m, n), jnp.float32),
    )(p0, p1, w)


def _add_body(a_ref, b_ref, o_ref):
    o_ref[...] = a_ref[...] + b_ref[...]


def _tc_add(a, b):
    m, n = a.shape
    return pl.pallas_call(
        _add_body,
        out_shape=jax.ShapeDtypeStruct((m, n), jnp.float32),
    )(a, b)


def _make_spmm(d):
    nvec = d // 16
    rps = 624
    tail = N_NODES - rps * NUM_SUB
    nfull = rps // CH
    rem = rps - nfull * CH
    nsteady = NCHUNKS // NTILES
    nleft = NCHUNKS - nsteady * NTILES
    mesh = plsc.VectorSubcoreMesh(core_axis_name="c", subcore_axis_name="s")
    cp = pltpu.CompilerParams(needs_layout_passes=False,
                              use_tc_tiling_on_sc=False)

    @functools.partial(
        pl.kernel,
        compiler_params=cp,
        out_type=jax.ShapeDtypeStruct((NUM_SC, N_NODES, d), jnp.float32),
        mesh=mesh,
        scratch_types=[
            pltpu.VMEM((3, CH), jnp.int32),
            pltpu.VMEM((3, CH), jnp.int32),
            pltpu.VMEM((CH, d), jnp.float32),
            pltpu.VMEM((CH, d), jnp.float32),
            pltpu.VMEM_SHARED((N_NODES, d), jnp.float32),
            pltpu.SemaphoreType.DMA,
            pltpu.SemaphoreType.DMA,
            pltpu.SemaphoreType.DMA,
            pltpu.SemaphoreType.DMA,
        ],
    )
    def spmm(h_hbm, evt_hbm, out_hbm,
             ebuf0, ebuf1, rows0, rows1, acc, semi0, semi1, semg0, semg1):
        cid = lax.axis_index("c")
        sid = lax.axis_index("s")
        wid = sid * NUM_SC + cid
        ebuf = (ebuf0, ebuf1)
        rows = (rows0, rows1)
        semi = (semi0, semi1)
        semg = (semg0, semg1)

        zero = jnp.zeros((16,), jnp.float32)

        @pl.loop(0, CH)
        def _(i):
            for j in range(nvec):
                rows0[i, pl.ds(j * 16, 16)] = zero

        base = sid * rps
        for k in range(nfull):
            pltpu.sync_copy(rows0, acc.at[pl.ds(base + k * CH, CH)])
        if rem:
            pltpu.sync_copy(rows0.at[pl.ds(0, rem)],
                            acc.at[pl.ds(base + nfull * CH, rem)])

        @pl.when(sid == NUM_SUB - 1)
        def _():
            pltpu.sync_copy(rows0.at[pl.ds(0, tail)],
                            acc.at[pl.ds(rps * NUM_SUB, tail)])

        plsc.subcore_barrier()

        zidx = jnp.zeros((16,), jnp.int32)
        two = jnp.full((16,), 2, jnp.int32)

        def idx_start(k, s):
            return pltpu.async_copy(evt_hbm.at[wid + k * NTILES],
                                    ebuf[s], semi[s])

        def gather_start(k, s):
            del k
            return pltpu.async_copy(h_hbm.at[ebuf[s].at[1]], rows[s], semg[s])

        def scale_and_scatter(s):
            @pl.loop(0, CH)
            def _(e):
                bc = plsc.bitcast(
                    plsc.load_gather(
                        ebuf[s], [two, jnp.full((16,), e, jnp.int32)]),
                    jnp.float32)
                for j in range(nvec):
                    sl = pl.ds(j * 16, 16)
                    rows[s][e, sl] = rows[s][e, sl] * bc

            pltpu.sync_copy(rows[s], acc.at[ebuf[s].at[0]], add=True)

        hidx = [None] * nsteady
        hgat = [None] * nsteady
        hidx[0] = idx_start(0, 0)
        if nsteady > 1:
            hidx[1] = idx_start(1, 1)
        hidx[0].wait()
        hgat[0] = gather_start(0, 0)
        for k in range(nsteady):
            s = k % 2
            if k + 1 < nsteady:
                hidx[k + 1].wait()
                hgat[k + 1] = gather_start(k + 1, 1 - s)
            hgat[k].wait()
            scale_and_scatter(s)
            if k + 2 < nsteady:
                hidx[k + 2] = idx_start(k + 2, s)

        @pl.when(wid < nleft)
        def _():
            pltpu.sync_copy(evt_hbm.at[wid + nsteady * NTILES], ebuf0)
            pltpu.sync_copy(h_hbm.at[ebuf0.at[1]], rows0)
            scale_and_scatter(0)

        plsc.subcore_barrier()
        pltpu.sync_copy(acc.at[pl.ds(base, rps)],
                        out_hbm.at[cid, pl.ds(base, rps)])

        @pl.when(sid == NUM_SUB - 1)
        def _():
            pltpu.sync_copy(acc.at[pl.ds(rps * NUM_SUB, tail)],
                            out_hbm.at[cid, pl.ds(rps * NUM_SUB, tail)])

    return spmm


_spmm128 = _make_spmm(128)
_spmm64 = _make_spmm(64)


def kernel(x, edge_index, adj_values, W1, W2):
    dst = edge_index[0].astype(jnp.int32).reshape(NCHUNKS, CH)
    src = edge_index[1].astype(jnp.int32).reshape(NCHUNKS, CH)
    vbits = jax.lax.bitcast_convert_type(
        adj_values, jnp.int32).reshape(NCHUNKS, CH)
    evt = jnp.stack([dst, src, vbits], axis=1)
    h1 = _tc_matmul(x, W1, bm=2000)
    p = _spmm128(h1, evt)
    h2 = _tc_add_relu_matmul(p[0], p[1], W2, bm=2000)
    q = _spmm64(h2, evt)
    return _tc_add(q[0], q[1])


# speedup vs baseline: 8.7205x; 1.3208x over previous
import dataclasses
import functools

import jax
import jax.numpy as jnp
from jax import lax
from jax.experimental import pallas as pl
from jax.experimental.pallas import tpu as pltpu
from jax.experimental.pallas import tpu_sc as plsc

N_NODES = 10000
E_EDGES = 160000
CH = 128
NCHUNKS = E_EDGES // CH
NUM_SC = 2
NUM_SUB = 16
NTILES = NUM_SC * NUM_SUB
ROWS_PER_SUB = N_NODES // NUM_SUB


def _matmul_body(x_ref, w_ref, o_ref):
    o_ref[...] = jnp.dot(x_ref[...], w_ref[...],
                         preferred_element_type=jnp.float32)


def _tc_matmul(x, w, bm):
    m, k = x.shape
    _, n = w.shape
    return pl.pallas_call(
        _matmul_body,
        grid=(m // bm,),
        in_specs=[pl.BlockSpec((bm, k), lambda i: (i, 0)),
                  pl.BlockSpec((k, n), lambda i: (0, 0))],
        out_specs=pl.BlockSpec((bm, n), lambda i: (i, 0)),
        out_shape=jax.ShapeDtypeStruct((m, n), jnp.float32),
    )(x, w)


def _fused_body(p0_ref, p1_ref, w_ref, o_ref):
    r = jnp.maximum(p0_ref[...] + p1_ref[...], 0.0)
    o_ref[...] = jnp.dot(r, w_ref[...], preferred_element_type=jnp.float32)


def _tc_add_relu_matmul(p0, p1, w, bm):
    m, k = p0.shape
    _, n = w.shape
    return pl.pallas_call(
        _fused_body,
        grid=(m // bm,),
        in_specs=[pl.BlockSpec((bm, k), lambda i: (i, 0)),
                  pl.BlockSpec((bm, k), lambda i: (i, 0)),
                  pl.BlockSpec((k, n), lambda i: (0, 0))],
        out_specs=pl.BlockSpec((bm, n), lambda i: (i, 0)),
        out_shape=jax.ShapeDtypeStruct((m, n), jnp.float32),
    )(p0, p1, w)


def _add_body(a_ref, b_ref, o_ref):
    o_ref[...] = a_ref[...] + b_ref[...]


def _tc_add(a, b):
    m, n = a.shape
    return pl.pallas_call(
        _add_body,
        out_shape=jax.ShapeDtypeStruct((m, n), jnp.float32),
    )(a, b)


def _make_spmm(d):
    nvec = d // 16
    rps = 624
    tail = N_NODES - rps * NUM_SUB
    nfull = rps // CH
    rem = rps - nfull * CH
    nsteady = NCHUNKS // NTILES
    nleft = NCHUNKS - nsteady * NTILES
    RS = 3 if d >= 128 else 4
    ES = 4
    mesh = plsc.VectorSubcoreMesh(core_axis_name="c", subcore_axis_name="s")
    cp = pltpu.CompilerParams(needs_layout_passes=False,
                              use_tc_tiling_on_sc=False)

    @functools.partial(
        pl.kernel,
        compiler_params=cp,
        out_type=jax.ShapeDtypeStruct((NUM_SC, N_NODES, d), jnp.float32),
        mesh=mesh,
        scratch_types=(
            [pltpu.VMEM((3, CH), jnp.int32) for _ in range(ES)]
            + [pltpu.VMEM((CH, d), jnp.float32) for _ in range(RS)]
            + [pltpu.VMEM_SHARED((N_NODES, d), jnp.float32)]
            + [pltpu.SemaphoreType.DMA for _ in range(ES + 2 * RS)]
        ),
    )
    def spmm(h_hbm, evt_hbm, out_hbm, *scr):
        ebuf = scr[0:ES]
        rows = scr[ES:ES + RS]
        acc = scr[ES + RS]
        sems = scr[ES + RS + 1:]
        semi = sems[0:ES]
        semg = sems[ES:ES + RS]
        sems_ = sems[ES + RS:ES + 2 * RS]
        rows0 = rows[0]
        eb0 = ebuf[0]
        ro0 = rows[0]
        cid = lax.axis_index("c")
        sid = lax.axis_index("s")
        wid = sid * NUM_SC + cid

        zero = jnp.zeros((16,), jnp.float32)

        @pl.loop(0, CH)
        def _(i):
            for j in range(nvec):
                rows0[i, pl.ds(j * 16, 16)] = zero

        base = sid * rps
        for k in range(nfull):
            pltpu.sync_copy(rows0, acc.at[pl.ds(base + k * CH, CH)])
        if rem:
            pltpu.sync_copy(rows0.at[pl.ds(0, rem)],
                            acc.at[pl.ds(base + nfull * CH, rem)])

        @pl.when(sid == NUM_SUB - 1)
        def _():
            pltpu.sync_copy(rows0.at[pl.ds(0, tail)],
                            acc.at[pl.ds(rps * NUM_SUB, tail)])

        plsc.subcore_barrier()

        zidx = jnp.zeros((16,), jnp.int32)
        two = jnp.full((16,), 2, jnp.int32)

        def idx_start(k):
            s = k % ES
            return pltpu.async_copy(evt_hbm.at[wid + k * NTILES],
                                    ebuf[s], semi[s])

        def gather_start(k):
            e, s = k % ES, k % RS
            return pltpu.async_copy(h_hbm.at[ebuf[e].at[1]], rows[s], semg[s])

        def scale(e, s):
            @plsc.parallel_loop(0, CH, unroll=4)
            def _(i):
                bc = plsc.bitcast(
                    plsc.load_gather(
                        ebuf[e], [two, jnp.full((16,), i, jnp.int32)]),
                    jnp.float32)
                for j in range(nvec):
                    sl = pl.ds(j * 16, 16)
                    rows[s][i, sl] = rows[s][i, sl] * bc

        def scatter_start(k):
            e, s = k % ES, k % RS
            return pltpu.async_copy(rows[s], acc.at[ebuf[e].at[0]], sems_[s],
                                    add=True)

        hidx = [None] * nsteady
        hgat = [None] * nsteady
        hsct = [None] * nsteady
        hidx[0] = idx_start(0)
        if nsteady > 1:
            hidx[1] = idx_start(1)
        hidx[0].wait()
        hgat[0] = gather_start(0)
        for k in range(nsteady):
            if k + 1 < nsteady:
                if k - 2 >= 0:
                    hsct[k - 2].wait()
                hidx[k + 1].wait()
                hgat[k + 1] = gather_start(k + 1)
            hgat[k].wait()
            scale(k % ES, k % RS)
            hsct[k] = scatter_start(k)
            if k + 2 < nsteady:
                hidx[k + 2] = idx_start(k + 2)
        for k in range(max(0, nsteady - 3), nsteady):
            if hsct[k] is not None and k + 3 >= nsteady:
                hsct[k].wait()

        @pl.when(wid < nleft)
        def _():
            pltpu.sync_copy(evt_hbm.at[wid + nsteady * NTILES], eb0)
            pltpu.sync_copy(h_hbm.at[eb0.at[1]], ro0)
            scale(0, 0)
            pltpu.sync_copy(ro0, acc.at[eb0.at[0]], add=True)

        plsc.subcore_barrier()
        pltpu.sync_copy(acc.at[pl.ds(base, rps)],
                        out_hbm.at[cid, pl.ds(base, rps)])

        @pl.when(sid == NUM_SUB - 1)
        def _():
            pltpu.sync_copy(acc.at[pl.ds(rps * NUM_SUB, tail)],
                            out_hbm.at[cid, pl.ds(rps * NUM_SUB, tail)])

    return spmm


_spmm128 = _make_spmm(128)
_spmm64 = _make_spmm(64)


def kernel(x, edge_index, adj_values, W1, W2):
    dst = edge_index[0].astype(jnp.int32).reshape(NCHUNKS, CH)
    src = edge_index[1].astype(jnp.int32).reshape(NCHUNKS, CH)
    vbits = jax.lax.bitcast_convert_type(
        adj_values, jnp.int32).reshape(NCHUNKS, CH)
    evt = jnp.stack([dst, src, vbits], axis=1)
    h1 = _tc_matmul(x, W1, bm=2000)
    p = _spmm128(h1, evt)
    h2 = _tc_add_relu_matmul(p[0], p[1], W2, bm=2000)
    q = _spmm64(h2, evt)
    return _tc_add(q[0], q[1])


# speedup vs baseline: 9.1701x; 1.0515x over previous
import dataclasses
import functools

import jax
import jax.numpy as jnp
from jax import lax
from jax.experimental import pallas as pl
from jax.experimental.pallas import tpu as pltpu
from jax.experimental.pallas import tpu_sc as plsc

N_NODES = 10000
E_EDGES = 160000
CH = 128
NCHUNKS = E_EDGES // CH
NUM_SC = 2
NUM_SUB = 16
NTILES = NUM_SC * NUM_SUB
ROWS_PER_SUB = N_NODES // NUM_SUB


def _matmul_body(x_ref, w_ref, o_ref):
    o_ref[...] = jnp.dot(x_ref[...], w_ref[...],
                         preferred_element_type=jnp.float32)


def _tc_matmul(x, w, bm):
    m, k = x.shape
    _, n = w.shape
    return pl.pallas_call(
        _matmul_body,
        grid=(m // bm,),
        in_specs=[pl.BlockSpec((bm, k), lambda i: (i, 0)),
                  pl.BlockSpec((k, n), lambda i: (0, 0))],
        out_specs=pl.BlockSpec((bm, n), lambda i: (i, 0)),
        out_shape=jax.ShapeDtypeStruct((m, n), jnp.float32),
    )(x, w)


def _fused_body(p0_ref, p1_ref, w_ref, o_ref):
    r = jnp.maximum(p0_ref[0] + p1_ref[0], 0.0)
    o_ref[...] = jnp.dot(r, w_ref[...], preferred_element_type=jnp.float32)


def _tc_add_relu_matmul(p, w, bm):
    _, m, k = p.shape
    _, n = w.shape
    return pl.pallas_call(
        _fused_body,
        grid=(m // bm,),
        in_specs=[pl.BlockSpec((1, bm, k), lambda i: (0, i, 0)),
                  pl.BlockSpec((1, bm, k), lambda i: (1, i, 0)),
                  pl.BlockSpec((k, n), lambda i: (0, 0))],
        out_specs=pl.BlockSpec((bm, n), lambda i: (i, 0)),
        out_shape=jax.ShapeDtypeStruct((m, n), jnp.float32),
    )(p, p, w)


def _add_body(a_ref, b_ref, o_ref):
    o_ref[...] = a_ref[0] + b_ref[0]


def _tc_add(q):
    _, m, n = q.shape
    return pl.pallas_call(
        _add_body,
        grid=(1,),
        in_specs=[pl.BlockSpec((1, m, n), lambda i: (0, 0, 0)),
                  pl.BlockSpec((1, m, n), lambda i: (1, 0, 0))],
        out_specs=pl.BlockSpec((m, n), lambda i: (0, 0)),
        out_shape=jax.ShapeDtypeStruct((m, n), jnp.float32),
    )(q, q)


def _make_spmm(d):
    nvec = d // 16
    rps = 624
    tail = N_NODES - rps * NUM_SUB
    nfull = rps // CH
    rem = rps - nfull * CH
    nsteady = NCHUNKS // NTILES
    nleft = NCHUNKS - nsteady * NTILES
    RS = 3 if d >= 128 else 4
    ES = 4
    mesh = plsc.VectorSubcoreMesh(core_axis_name="c", subcore_axis_name="s")
    cp = pltpu.CompilerParams(needs_layout_passes=False,
                              use_tc_tiling_on_sc=False)

    @functools.partial(
        pl.kernel,
        compiler_params=cp,
        out_type=jax.ShapeDtypeStruct((NUM_SC, N_NODES, d), jnp.float32),
        mesh=mesh,
        scratch_types=(
            [pltpu.VMEM((2, CH), jnp.int32) for _ in range(ES)]
            + [pltpu.VMEM((1, CH), jnp.float32) for _ in range(ES)]
            + [pltpu.VMEM((CH, d), jnp.float32) for _ in range(RS)]
            + [pltpu.VMEM_SHARED((N_NODES, d), jnp.float32)]
            + [pltpu.SemaphoreType.DMA for _ in range(ES + 2 * RS)]
        ),
    )
    def spmm(h_hbm, dst_hbm, src_hbm, val_hbm, out_hbm, *scr):
        ebuf = scr[0:ES]
        vbuf = scr[ES:2 * ES]
        rows = scr[2 * ES:2 * ES + RS]
        acc = scr[2 * ES + RS]
        sems = scr[2 * ES + RS + 1:]
        semi = sems[0:ES]
        semg = sems[ES:ES + RS]
        sems_ = sems[ES + RS:ES + 2 * RS]
        rows0 = rows[0]
        eb0 = ebuf[0]
        ro0 = rows[0]
        cid = lax.axis_index("c")
        sid = lax.axis_index("s")
        wid = sid * NUM_SC + cid

        zero = jnp.zeros((16,), jnp.float32)

        @pl.loop(0, CH)
        def _(i):
            for j in range(nvec):
                rows0[i, pl.ds(j * 16, 16)] = zero

        base = sid * rps
        for k in range(nfull):
            pltpu.sync_copy(rows0, acc.at[pl.ds(base + k * CH, CH)])
        if rem:
            pltpu.sync_copy(rows0.at[pl.ds(0, rem)],
                            acc.at[pl.ds(base + nfull * CH, rem)])

        @pl.when(sid == NUM_SUB - 1)
        def _():
            pltpu.sync_copy(rows0.at[pl.ds(0, tail)],
                            acc.at[pl.ds(rps * NUM_SUB, tail)])

        plsc.subcore_barrier()

        zidx = jnp.zeros((16,), jnp.int32)

        class _Handles:
            def __init__(self, hs):
                self.hs = hs

            def wait(self):
                for h in self.hs:
                    h.wait()

        def idx_start(k):
            s = k % ES
            c = wid + k * NTILES
            return _Handles([
                pltpu.async_copy(dst_hbm.at[c], ebuf[s].at[0], semi[s]),
                pltpu.async_copy(src_hbm.at[c], ebuf[s].at[1], semi[s]),
                pltpu.async_copy(val_hbm.at[c], vbuf[s].at[0], semi[s]),
            ])

        def gather_start(k):
            e, s = k % ES, k % RS
            return pltpu.async_copy(h_hbm.at[ebuf[e].at[1]], rows[s], semg[s])

        def scale(e, s):
            @plsc.parallel_loop(0, CH, unroll=4)
            def _(i):
                bc = plsc.load_gather(
                    vbuf[e], [zidx, jnp.full((16,), i, jnp.int32)])
                for j in range(nvec):
                    sl = pl.ds(j * 16, 16)
                    rows[s][i, sl] = rows[s][i, sl] * bc

        def scatter_start(k):
            e, s = k % ES, k % RS
            return pltpu.async_copy(rows[s], acc.at[ebuf[e].at[0]], sems_[s],
                                    add=True)

        hidx = [None] * nsteady
        hgat = [None] * nsteady
        hsct = [None] * nsteady
        hidx[0] = idx_start(0)
        if nsteady > 1:
            hidx[1] = idx_start(1)
        hidx[0].wait()
        hgat[0] = gather_start(0)
        for k in range(nsteady):
            if k + 1 < nsteady:
                if k - 2 >= 0:
                    hsct[k - 2].wait()
                hidx[k + 1].wait()
                hgat[k + 1] = gather_start(k + 1)
            hgat[k].wait()
            scale(k % ES, k % RS)
            hsct[k] = scatter_start(k)
            if k + 2 < nsteady:
                hidx[k + 2] = idx_start(k + 2)
        for k in range(max(0, nsteady - 3), nsteady):
            if hsct[k] is not None and k + 3 >= nsteady:
                hsct[k].wait()

        @pl.when(wid < nleft)
        def _():
            c = wid + nsteady * NTILES
            pltpu.sync_copy(dst_hbm.at[c], eb0.at[0])
            pltpu.sync_copy(src_hbm.at[c], eb0.at[1])
            pltpu.sync_copy(val_hbm.at[c], vbuf[0].at[0])
            pltpu.sync_copy(h_hbm.at[eb0.at[1]], ro0)
            scale(0, 0)
            pltpu.sync_copy(ro0, acc.at[eb0.at[0]], add=True)

        plsc.subcore_barrier()
        pltpu.sync_copy(acc.at[pl.ds(base, rps)],
                        out_hbm.at[cid, pl.ds(base, rps)])

        @pl.when(sid == NUM_SUB - 1)
        def _():
            pltpu.sync_copy(acc.at[pl.ds(rps * NUM_SUB, tail)],
                            out_hbm.at[cid, pl.ds(rps * NUM_SUB, tail)])

    return spmm


_spmm128 = _make_spmm(128)
_spmm64 = _make_spmm(64)


def kernel(x, edge_index, adj_values, W1, W2):
    dst = edge_index[0].astype(jnp.int32).reshape(NCHUNKS, CH)
    src = edge_index[1].astype(jnp.int32).reshape(NCHUNKS, CH)
    val = adj_values.reshape(NCHUNKS, CH)
    h1 = _tc_matmul(x, W1, bm=2000)
    p = _spmm128(h1, dst, src, val)
    h2 = _tc_add_relu_matmul(p, W2, bm=2000)
    q = _spmm64(h2, dst, src, val)
    return _tc_add(q)


# speedup vs baseline: 9.2204x; 1.0055x over previous
import dataclasses
import functools

import jax
import jax.numpy as jnp
from jax import lax
from jax.experimental import pallas as pl
from jax.experimental.pallas import tpu as pltpu
from jax.experimental.pallas import tpu_sc as plsc

N_NODES = 10000
E_EDGES = 160000
CH = 128
NCHUNKS = E_EDGES // CH
NUM_SC = 2
NUM_SUB = 16
NTILES = NUM_SC * NUM_SUB
ROWS_PER_SUB = N_NODES // NUM_SUB


def _matmul_body(x_ref, w_ref, o_ref):
    o_ref[...] = jnp.dot(x_ref[...], w_ref[...],
                         preferred_element_type=jnp.float32)


def _tc_matmul(x, w, bm):
    m, k = x.shape
    _, n = w.shape
    return pl.pallas_call(
        _matmul_body,
        grid=(m // bm,),
        in_specs=[pl.BlockSpec((bm, k), lambda i: (i, 0)),
                  pl.BlockSpec((k, n), lambda i: (0, 0))],
        out_specs=pl.BlockSpec((bm, n), lambda i: (i, 0)),
        out_shape=jax.ShapeDtypeStruct((m, n), jnp.float32),
    )(x, w)


def _fused_body(p0_ref, p1_ref, w_ref, o_ref):
    r = jnp.maximum(p0_ref[0] + p1_ref[0], 0.0)
    o_ref[...] = jnp.dot(r, w_ref[...], preferred_element_type=jnp.float32)


def _tc_add_relu_matmul(p, w, bm):
    _, m, k = p.shape
    _, n = w.shape
    return pl.pallas_call(
        _fused_body,
        grid=(m // bm,),
        in_specs=[pl.BlockSpec((1, bm, k), lambda i: (0, i, 0)),
                  pl.BlockSpec((1, bm, k), lambda i: (1, i, 0)),
                  pl.BlockSpec((k, n), lambda i: (0, 0))],
        out_specs=pl.BlockSpec((bm, n), lambda i: (i, 0)),
        out_shape=jax.ShapeDtypeStruct((m, n), jnp.float32),
    )(p, p, w)


def _add_body(a_ref, b_ref, o_ref):
    o_ref[...] = a_ref[0] + b_ref[0]


def _tc_add(q):
    _, m, n = q.shape
    return pl.pallas_call(
        _add_body,
        grid=(1,),
        in_specs=[pl.BlockSpec((1, m, n), lambda i: (0, 0, 0)),
                  pl.BlockSpec((1, m, n), lambda i: (1, 0, 0))],
        out_specs=pl.BlockSpec((m, n), lambda i: (0, 0)),
        out_shape=jax.ShapeDtypeStruct((m, n), jnp.float32),
    )(q, q)


def _make_spmm(d):
    nvec = d // 16
    rps = 624
    tail = N_NODES - rps * NUM_SUB
    nfull = rps // CH
    rem = rps - nfull * CH
    nsteady = NCHUNKS // NTILES
    nleft = NCHUNKS - nsteady * NTILES
    RS = 3 if d >= 128 else 4
    ES = 4
    mesh = plsc.VectorSubcoreMesh(core_axis_name="c", subcore_axis_name="s")
    cp = pltpu.CompilerParams(needs_layout_passes=False,
                              use_tc_tiling_on_sc=False)

    @functools.partial(
        pl.kernel,
        compiler_params=cp,
        out_type=jax.ShapeDtypeStruct((NUM_SC, N_NODES, d), jnp.float32),
        mesh=mesh,
        scratch_types=(
            [pltpu.VMEM((2, CH), jnp.int32) for _ in range(ES)]
            + [pltpu.VMEM((1, CH), jnp.float32) for _ in range(ES)]
            + [pltpu.VMEM((CH, d), jnp.float32) for _ in range(RS)]
            + [pltpu.VMEM_SHARED((N_NODES, d), jnp.float32)]
            + [pltpu.SemaphoreType.DMA for _ in range(ES + 2 * RS)]
        ),
    )
    def spmm(h_hbm, dst_hbm, src_hbm, val_hbm, out_hbm, *scr):
        ebuf = scr[0:ES]
        vbuf = scr[ES:2 * ES]
        rows = scr[2 * ES:2 * ES + RS]
        acc = scr[2 * ES + RS]
        sems = scr[2 * ES + RS + 1:]
        semi = sems[0:ES]
        semg = sems[ES:ES + RS]
        sems_ = sems[ES + RS:ES + 2 * RS]
        rows0 = rows[0]
        eb0 = ebuf[0]
        ro0 = rows[0]
        cid = lax.axis_index("c")
        sid = lax.axis_index("s")
        wid = sid * NUM_SC + cid

        zidx = jnp.zeros((16,), jnp.int32)

        class _Handles:
            def __init__(self, hs):
                self.hs = hs

            def wait(self):
                for h in self.hs:
                    h.wait()

        def idx_start(k):
            s = k % ES
            e0 = (wid + k * NTILES) * CH
            return _Handles([
                pltpu.async_copy(dst_hbm.at[pl.ds(e0, CH)], ebuf[s].at[0],
                                 semi[s]),
                pltpu.async_copy(src_hbm.at[pl.ds(e0, CH)], ebuf[s].at[1],
                                 semi[s]),
                pltpu.async_copy(val_hbm.at[pl.ds(e0, CH)], vbuf[s].at[0],
                                 semi[s]),
            ])

        def gather_start(k):
            e, s = k % ES, k % RS
            return pltpu.async_copy(h_hbm.at[ebuf[e].at[1]], rows[s], semg[s])

        def scale(e, s):
            @plsc.parallel_loop(0, CH, unroll=4)
            def _(i):
                bc = plsc.load_gather(
                    vbuf[e], [zidx, jnp.full((16,), i, jnp.int32)])
                for j in range(nvec):
                    sl = pl.ds(j * 16, 16)
                    rows[s][i, sl] = rows[s][i, sl] * bc

        def scatter_start(k):
            e, s = k % ES, k % RS
            return pltpu.async_copy(rows[s], acc.at[ebuf[e].at[0]], sems_[s],
                                    add=True)

        hidx = [None] * nsteady
        hgat = [None] * nsteady
        hsct = [None] * nsteady
        hidx[0] = idx_start(0)
        if nsteady > 1:
            hidx[1] = idx_start(1)

        zsrc = rows[RS - 1]
        zero = jnp.zeros((16,), jnp.float32)

        @pl.loop(0, CH)
        def _(i):
            for j in range(nvec):
                zsrc[i, pl.ds(j * 16, 16)] = zero

        base = sid * rps
        for k in range(nfull):
            pltpu.sync_copy(zsrc, acc.at[pl.ds(base + k * CH, CH)])
        if rem:
            pltpu.sync_copy(zsrc.at[pl.ds(0, rem)],
                            acc.at[pl.ds(base + nfull * CH, rem)])

        @pl.when(sid == NUM_SUB - 1)
        def _():
            pltpu.sync_copy(zsrc.at[pl.ds(0, tail)],
                            acc.at[pl.ds(rps * NUM_SUB, tail)])

        hidx[0].wait()
        hgat[0] = gather_start(0)
        plsc.subcore_barrier()

        for k in range(nsteady):
            if k + 1 < nsteady:
                if k - 2 >= 0:
                    hsct[k - 2].wait()
                hidx[k + 1].wait()
                hgat[k + 1] = gather_start(k + 1)
            hgat[k].wait()
            scale(k % ES, k % RS)
            hsct[k] = scatter_start(k)
            if k + 2 < nsteady:
                hidx[k + 2] = idx_start(k + 2)
        for k in range(max(0, nsteady - 3), nsteady):
            if hsct[k] is not None and k + 3 >= nsteady:
                hsct[k].wait()

        @pl.when(wid < nleft)
        def _():
            e0 = (wid + nsteady * NTILES) * CH
            pltpu.sync_copy(dst_hbm.at[pl.ds(e0, CH)], eb0.at[0])
            pltpu.sync_copy(src_hbm.at[pl.ds(e0, CH)], eb0.at[1])
            pltpu.sync_copy(val_hbm.at[pl.ds(e0, CH)], vbuf[0].at[0])
            pltpu.sync_copy(h_hbm.at[eb0.at[1]], ro0)
            scale(0, 0)
            pltpu.sync_copy(ro0, acc.at[eb0.at[0]], add=True)

        plsc.subcore_barrier()
        pltpu.sync_copy(acc.at[pl.ds(base, rps)],
                        out_hbm.at[cid, pl.ds(base, rps)])

        @pl.when(sid == NUM_SUB - 1)
        def _():
            pltpu.sync_copy(acc.at[pl.ds(rps * NUM_SUB, tail)],
                            out_hbm.at[cid, pl.ds(rps * NUM_SUB, tail)])

    return spmm


_spmm128 = _make_spmm(128)
_spmm64 = _make_spmm(64)


def kernel(x, edge_index, adj_values, W1, W2):
    dst = edge_index[0].astype(jnp.int32)
    src = edge_index[1].astype(jnp.int32)
    val = adj_values
    h1 = _tc_matmul(x, W1, bm=2000)
    p = _spmm128(h1, dst, src, val)
    h2 = _tc_add_relu_matmul(p, W2, bm=2000)
    q = _spmm64(h2, dst, src, val)
    return _tc_add(q)


# speedup vs baseline: 9.7618x; 1.0587x over previous
import dataclasses
import functools

import jax
import jax.numpy as jnp
from jax import lax
from jax.experimental import pallas as pl
from jax.experimental.pallas import tpu as pltpu
from jax.experimental.pallas import tpu_sc as plsc

N_NODES = 10000
E_EDGES = 160000
CH = 128
NCHUNKS = E_EDGES // CH
NUM_SC = 2
NUM_SUB = 16
NTILES = NUM_SC * NUM_SUB
ROWS_PER_SUB = N_NODES // NUM_SUB


def _matmul_body(x_ref, w_ref, o_ref):
    o_ref[...] = jnp.dot(x_ref[...], w_ref[...],
                         preferred_element_type=jnp.float32)


def _tc_matmul(x, w, bm):
    m, k = x.shape
    _, n = w.shape
    return pl.pallas_call(
        _matmul_body,
        grid=(m // bm,),
        in_specs=[pl.BlockSpec((bm, k), lambda i: (i, 0)),
                  pl.BlockSpec((k, n), lambda i: (0, 0))],
        out_specs=pl.BlockSpec((bm, n), lambda i: (i, 0)),
        out_shape=jax.ShapeDtypeStruct((m, n), jnp.float32),
    )(x, w)


def _fused_body(p0_ref, p1_ref, w_ref, o_ref):
    r = jnp.maximum(p0_ref[0] + p1_ref[0], 0.0)
    o_ref[...] = jnp.dot(r, w_ref[...], preferred_element_type=jnp.float32)


def _tc_add_relu_matmul(p, w, bm):
    _, m, k = p.shape
    _, n = w.shape
    return pl.pallas_call(
        _fused_body,
        grid=(m // bm,),
        in_specs=[pl.BlockSpec((1, bm, k), lambda i: (0, i, 0)),
                  pl.BlockSpec((1, bm, k), lambda i: (1, i, 0)),
                  pl.BlockSpec((k, n), lambda i: (0, 0))],
        out_specs=pl.BlockSpec((bm, n), lambda i: (i, 0)),
        out_shape=jax.ShapeDtypeStruct((m, n), jnp.float32),
    )(p, p, w)


def _add_body(a_ref, b_ref, o_ref):
    o_ref[...] = (a_ref[0] + b_ref[0]).T


def _tc_add_t(q):
    _, m, n = q.shape
    return pl.pallas_call(
        _add_body,
        grid=(1,),
        in_specs=[pl.BlockSpec((1, m, n), lambda i: (0, 0, 0)),
                  pl.BlockSpec((1, m, n), lambda i: (1, 0, 0))],
        out_specs=pl.BlockSpec((n, m), lambda i: (0, 0)),
        out_shape=jax.ShapeDtypeStruct((n, m), jnp.float32),
    )(q, q)


def _make_spmm(d):
    nvec = d // 16
    rps = 624
    tail = N_NODES - rps * NUM_SUB
    nfull = rps // CH
    rem = rps - nfull * CH
    nsteady = NCHUNKS // NTILES
    nleft = NCHUNKS - nsteady * NTILES
    RS = 3 if d >= 128 else 4
    ES = 4
    mesh = plsc.VectorSubcoreMesh(core_axis_name="c", subcore_axis_name="s")
    cp = pltpu.CompilerParams(needs_layout_passes=False,
                              use_tc_tiling_on_sc=False)

    @functools.partial(
        pl.kernel,
        compiler_params=cp,
        out_type=jax.ShapeDtypeStruct((NUM_SC, N_NODES, d), jnp.float32),
        mesh=mesh,
        scratch_types=(
            [pltpu.VMEM((2, CH), jnp.int32) for _ in range(ES)]
            + [pltpu.VMEM((1, CH), jnp.float32) for _ in range(ES)]
            + [pltpu.VMEM((CH, d), jnp.float32) for _ in range(RS)]
            + [pltpu.VMEM_SHARED((N_NODES, d), jnp.float32)]
            + [pltpu.SemaphoreType.DMA for _ in range(ES + 2 * RS)]
        ),
    )
    def spmm(h_hbm, ei_hbm, val_hbm, out_hbm, *scr):
        ebuf = scr[0:ES]
        vbuf = scr[ES:2 * ES]
        rows = scr[2 * ES:2 * ES + RS]
        acc = scr[2 * ES + RS]
        sems = scr[2 * ES + RS + 1:]
        semi = sems[0:ES]
        semg = sems[ES:ES + RS]
        sems_ = sems[ES + RS:ES + 2 * RS]
        rows0 = rows[0]
        eb0 = ebuf[0]
        ro0 = rows[0]
        cid = lax.axis_index("c")
        sid = lax.axis_index("s")
        wid = sid * NUM_SC + cid

        zidx = jnp.zeros((16,), jnp.int32)

        class _Handles:
            def __init__(self, hs):
                self.hs = hs

            def wait(self):
                for h in self.hs:
                    h.wait()

        def idx_start(k):
            s = k % ES
            e0 = (wid + k * NTILES) * CH
            return _Handles([
                pltpu.async_copy(ei_hbm.at[0, pl.ds(e0, CH)], ebuf[s].at[0],
                                 semi[s]),
                pltpu.async_copy(ei_hbm.at[1, pl.ds(e0, CH)], ebuf[s].at[1],
                                 semi[s]),
                pltpu.async_copy(val_hbm.at[pl.ds(e0, CH)], vbuf[s].at[0],
                                 semi[s]),
            ])

        def gather_start(k):
            e, s = k % ES, k % RS
            return pltpu.async_copy(h_hbm.at[ebuf[e].at[1]], rows[s], semg[s])

        def scale(e, s):
            @plsc.parallel_loop(0, CH, unroll=4)
            def _(i):
                bc = plsc.load_gather(
                    vbuf[e], [zidx, jnp.full((16,), i, jnp.int32)])
                for j in range(nvec):
                    sl = pl.ds(j * 16, 16)
                    rows[s][i, sl] = rows[s][i, sl] * bc

        def scatter_start(k):
            e, s = k % ES, k % RS
            return pltpu.async_copy(rows[s], acc.at[ebuf[e].at[0]], sems_[s],
                                    add=True)

        hidx = [None] * nsteady
        hgat = [None] * nsteady
        hsct = [None] * nsteady
        hidx[0] = idx_start(0)
        if nsteady > 1:
            hidx[1] = idx_start(1)

        zsrc = rows[RS - 1]
        zero = jnp.zeros((16,), jnp.float32)

        @pl.loop(0, CH)
        def _(i):
            for j in range(nvec):
                zsrc[i, pl.ds(j * 16, 16)] = zero

        base = sid * rps
        for k in range(nfull):
            pltpu.sync_copy(zsrc, acc.at[pl.ds(base + k * CH, CH)])
        if rem:
            pltpu.sync_copy(zsrc.at[pl.ds(0, rem)],
                            acc.at[pl.ds(base + nfull * CH, rem)])

        @pl.when(sid == NUM_SUB - 1)
        def _():
            pltpu.sync_copy(zsrc.at[pl.ds(0, tail)],
                            acc.at[pl.ds(rps * NUM_SUB, tail)])

        hidx[0].wait()
        hgat[0] = gather_start(0)
        plsc.subcore_barrier()

        for k in range(nsteady):
            if k + 1 < nsteady:
                if k - 2 >= 0:
                    hsct[k - 2].wait()
                hidx[k + 1].wait()
                hgat[k + 1] = gather_start(k + 1)
            hgat[k].wait()
            scale(k % ES, k % RS)
            hsct[k] = scatter_start(k)
            if k + 2 < nsteady:
                hidx[k + 2] = idx_start(k + 2)
        for k in range(max(0, nsteady - 3), nsteady):
            if hsct[k] is not None and k + 3 >= nsteady:
                hsct[k].wait()

        @pl.when(wid < nleft)
        def _():
            e0 = (wid + nsteady * NTILES) * CH
            pltpu.sync_copy(ei_hbm.at[0, pl.ds(e0, CH)], eb0.at[0])
            pltpu.sync_copy(ei_hbm.at[1, pl.ds(e0, CH)], eb0.at[1])
            pltpu.sync_copy(val_hbm.at[pl.ds(e0, CH)], vbuf[0].at[0])
            pltpu.sync_copy(h_hbm.at[eb0.at[1]], ro0)
            scale(0, 0)
            pltpu.sync_copy(ro0, acc.at[eb0.at[0]], add=True)

        plsc.subcore_barrier()
        pltpu.sync_copy(acc.at[pl.ds(base, rps)],
                        out_hbm.at[cid, pl.ds(base, rps)])

        @pl.when(sid == NUM_SUB - 1)
        def _():
            pltpu.sync_copy(acc.at[pl.ds(rps * NUM_SUB, tail)],
                            out_hbm.at[cid, pl.ds(rps * NUM_SUB, tail)])

    return spmm


_spmm128 = _make_spmm(128)
_spmm64 = _make_spmm(64)


def kernel(x, edge_index, adj_values, W1, W2):
    ei = edge_index.astype(jnp.int32)
    h1 = _tc_matmul(x, W1, bm=2000)
    p = _spmm128(h1, ei, adj_values)
    h2 = _tc_add_relu_matmul(p, W2, bm=2000)
    q = _spmm64(h2, ei, adj_values)
    return _tc_add_t(q).T
